# baseline (speedup 1.0000x reference)
import functools

import jax
import jax.numpy as jnp
from jax import lax
from jax.experimental import pallas as pl
from jax.experimental.pallas import tpu as pltpu
from jax.experimental.pallas import tpu_sc as plsc

N_CATEGORIES = 1000000
D = 32
B = 16384
NC, NS, L = 2, 16, 16
NW = NC * NS
BPW = B // NW
CHUNK = 128
NCHUNK = BPW // CHUNK


def _rsqrt(x):
    i = lax.bitcast_convert_type(x, jnp.int32)
    i = jnp.int32(0x5F3759DF) - (i >> 1)
    y = lax.bitcast_convert_type(i, jnp.float32)
    xh = x * jnp.float32(-0.5)
    for _ in range(3):
        y = y * (jnp.float32(1.5) + xh * y * y)
    return y


def _encoder_body(idx_hbm, tab_hbm, out_hbm, idx_v, cols_v, sem):
    wid = lax.axis_index("s") * NC + lax.axis_index("c")
    base = wid * BPW

    pltpu.sync_copy(idx_hbm.at[pl.ds(base, BPW)], idx_v)

    for c in range(D):
        for j in range(NCHUNK):
            pltpu.async_copy(
                tab_hbm.at[c].at[idx_v.at[pl.ds(j * CHUNK, CHUNK)]],
                cols_v.at[c, pl.ds(j * CHUNK, CHUNK)],
                sem,
            )
    pltpu.make_async_copy(tab_hbm.at[:, pl.ds(0, BPW)], cols_v, sem).wait()

    def group_body(g, carry):
        o = g * L
        vs = [cols_v[c, pl.ds(o, L)] for c in range(D)]
        ss = None
        for v in vs:
            ss = v * v if ss is None else ss + v * v
        inv = _rsqrt(ss)
        for c in range(D):
            cols_v[c, pl.ds(o, L)] = vs[c] * inv
        return carry

    lax.fori_loop(0, BPW // L, group_body, 0)

    pltpu.sync_copy(cols_v, out_hbm.at[:, pl.ds(base, BPW)])


_encoder = functools.partial(
    pl.kernel,
    out_type=jax.ShapeDtypeStruct((D, B), jnp.float32),
    mesh=plsc.VectorSubcoreMesh(core_axis_name="c", subcore_axis_name="s"),
    compiler_params=pltpu.CompilerParams(
        needs_layout_passes=False, use_tc_tiling_on_sc=False
    ),
    scratch_types=[
        pltpu.VMEM((BPW,), jnp.int32),
        pltpu.VMEM((D, BPW), jnp.float32),
        pltpu.SemaphoreType.DMA,
    ],
)(_encoder_body)


def kernel(src, categories_means, categories_logvars):
    del categories_logvars
    idx = src.astype(jnp.int32)
    out_t = _encoder(idx, categories_means.T)
    return out_t.T


# speedup vs baseline: 1.0009x; 1.0009x over previous
import functools

import jax
import jax.numpy as jnp
from jax import lax
from jax.experimental import pallas as pl
from jax.experimental.pallas import tpu as pltpu
from jax.experimental.pallas import tpu_sc as plsc

N_CATEGORIES = 1000000
D = 32
B = 16384
NC, NS, L = 2, 16, 16
NW = NC * NS
BPW = B // NW
CHUNK = 128
NCHUNK = BPW // CHUNK


def _rsqrt(x):
    i = lax.bitcast_convert_type(x, jnp.int32)
    i = jnp.int32(0x5F3759DF) - (i >> 1)
    y = lax.bitcast_convert_type(i, jnp.float32)
    xh = x * jnp.float32(-0.5)
    for _ in range(3):
        y = y * (jnp.float32(1.5) + xh * y * y)
    return y


def _encoder_body(idx_hbm, tab_hbm, out_hbm, idx_v, cols_v, sem):
    wid = lax.axis_index("s") * NC + lax.axis_index("c")
    base = wid * BPW

    pltpu.sync_copy(idx_hbm.at[pl.ds(base, BPW)], idx_v)

    for c in range(D):
        for j in range(NCHUNK):
            pltpu.async_copy(
                tab_hbm.at[c].at[idx_v.at[pl.ds(j * CHUNK, CHUNK)]],
                cols_v.at[c, pl.ds(j * CHUNK, CHUNK)],
                sem,
            )
    pltpu.make_async_copy(tab_hbm.at[:, pl.ds(0, BPW)], cols_v, sem).wait()

    def group_body(g, carry):
        o = g * L
        vs = [cols_v[c, pl.ds(o, L)] for c in range(D)]
        ss = None
        for v in vs:
            ss = v * v if ss is None else ss + v * v
        inv = _rsqrt(ss)
        for c in range(D):
            cols_v[c, pl.ds(o, L)] = vs[c] * inv
        return carry

    lax.fori_loop(0, BPW // L, group_body, 0)

    pltpu.sync_copy(cols_v, out_hbm.at[:, pl.ds(base, BPW)])


_encoder = functools.partial(
    pl.kernel,
    out_type=jax.ShapeDtypeStruct((D, B), jnp.float32),
    mesh=plsc.VectorSubcoreMesh(core_axis_name="c", subcore_axis_name="s"),
    compiler_params=pltpu.CompilerParams(
        needs_layout_passes=True, use_tc_tiling_on_sc=False
    ),
    scratch_types=[
        pltpu.VMEM((BPW,), jnp.int32),
        pltpu.VMEM((D, BPW), jnp.float32),
        pltpu.SemaphoreType.DMA,
    ],
)(_encoder_body)


def kernel(src, categories_means, categories_logvars):
    del categories_logvars
    idx = src.astype(jnp.int32)
    out_t = _encoder(idx, categories_means.T)
    return out_t.T


# speedup vs baseline: 7.8654x; 7.8587x over previous
import functools

import jax
import jax.numpy as jnp
from jax import lax
from jax.experimental import pallas as pl
from jax.experimental.pallas import tpu as pltpu
from jax.experimental.pallas import tpu_sc as plsc

N_CATEGORIES = 1000000
D = 32
B = 16384
NC, NS, L = 2, 16, 16
NW = NC * NS
BPW = B // NW
ROWTILES = BPW // L


def _rsqrt(x):
    i = lax.bitcast_convert_type(x, jnp.int32)
    i = jnp.int32(0x5F3759DF) - (i >> 1)
    y = lax.bitcast_convert_type(i, jnp.float32)
    xh = x * jnp.float32(-0.5)
    for _ in range(3):
        y = y * (jnp.float32(1.5) + xh * y * y)
    return y


def _encoder_body(idx_hbm, table_hbm, out_hbm, idx_v, rows_v, sem):
    wid = lax.axis_index("s") * NC + lax.axis_index("c")
    base = wid * BPW

    pltpu.sync_copy(idx_hbm.at[pl.ds(base, BPW)], idx_v)

    def fire(g, carry):
        iv = idx_v[pl.ds(g * L, L)]
        for j in range(L):
            pltpu.async_copy(table_hbm.at[iv[j]], rows_v.at[g * L + j], sem)
        return carry

    lax.fori_loop(0, BPW // L, fire, 0)
    pltpu.make_async_copy(table_hbm.at[pl.ds(0, BPW)], rows_v, sem).wait()

    lane = lax.iota(jnp.int32, L)

    def tile_body(t, carry):
        rid = t * L + lane
        cols = []
        ss = None
        for j in range(D):
            cid = jnp.full((L,), j, jnp.int32)
            cj = plsc.load_gather(rows_v, [rid, cid])
            cols.append(cj)
            ss = cj * cj if ss is None else ss + cj * cj
        inv = _rsqrt(ss)
        for j in range(D):
            cid = jnp.full((L,), j, jnp.int32)
            plsc.store_scatter(rows_v, [rid, cid], cols[j] * inv)
        return carry

    lax.fori_loop(0, ROWTILES, tile_body, 0)

    pltpu.sync_copy(rows_v, out_hbm.at[pl.ds(base, BPW)])


_encoder = functools.partial(
    pl.kernel,
    out_type=jax.ShapeDtypeStruct((B, D), jnp.float32),
    mesh=plsc.VectorSubcoreMesh(core_axis_name="c", subcore_axis_name="s"),
    compiler_params=pltpu.CompilerParams(needs_layout_passes=False),
    scratch_types=[
        pltpu.VMEM((BPW,), jnp.int32),
        pltpu.VMEM((BPW, D), jnp.float32),
        pltpu.SemaphoreType.DMA,
    ],
)(_encoder_body)


def kernel(src, categories_means, categories_logvars):
    del categories_logvars
    return _encoder(src.astype(jnp.int32), categories_means)
